# Initial kernel scaffold; baseline (speedup 1.0000x reference)
import jax
import jax.numpy as jnp
from jax.experimental import pallas as pl


def kernel(y_true, y_pred):
    raise NotImplementedError("write your pallas kernel here")


# speedup vs baseline: 1.3806x; 1.3806x over previous
import jax
import jax.numpy as jnp
from jax.experimental import pallas as pl
from jax.experimental.pallas import tpu as pltpu

_C = 8
_LANES = 128
_RPL = _LANES // _C
_BR = 4096
_NCORES = 2


def _segmented_onehot(x):
    q = jax.lax.broadcasted_iota(jnp.int32, (1, _LANES), 1) % _C
    m = x
    for d in (4, 2, 1):
        lo = pltpu.roll(m, _LANES - d, axis=1)
        hi = pltpu.roll(m, d, axis=1)
        partner = jnp.where((q & d) == 0, lo, hi)
        m = jnp.maximum(m, partner)
    return jnp.where(x == m, 1.0, 0.0).astype(jnp.bfloat16)


def _cm_kernel(xt_ref, xp_ref, out_ref):
    j = pl.program_id(1)
    oh_t = _segmented_onehot(xt_ref[...])
    oh_p = _segmented_onehot(xp_ref[...])
    g = jax.lax.dot_general(
        oh_t, oh_p, (((0,), (0,)), ((), ())),
        preferred_element_type=jnp.float32,
    )

    @pl.when(j == 0)
    def _():
        out_ref[...] = jnp.zeros_like(out_ref)

    out_ref[...] += g


def kernel(y_true, y_pred):
    n = y_true.shape[0]
    r = n // _RPL
    xt = y_true.reshape(r, _LANES)
    xp = y_pred.reshape(r, _LANES)
    k = r // (_BR * _NCORES)
    grid = (_NCORES, k)
    g = pl.pallas_call(
        _cm_kernel,
        grid=grid,
        in_specs=[
            pl.BlockSpec((_BR, _LANES), lambda i, j: (i * k + j, 0)),
            pl.BlockSpec((_BR, _LANES), lambda i, j: (i * k + j, 0)),
        ],
        out_specs=pl.BlockSpec((None, _LANES, _LANES), lambda i, j: (i, 0, 0)),
        out_shape=jax.ShapeDtypeStruct((_NCORES, _LANES, _LANES), jnp.float32),
        compiler_params=pltpu.CompilerParams(
            dimension_semantics=("parallel", "arbitrary"),
        ),
    )(xt, xp)
    g2 = g.sum(axis=0).reshape(_RPL, _C, _RPL, _C)
    return jnp.einsum("sasb->ab", g2)


# speedup vs baseline: 33.0848x; 23.9633x over previous
import jax
import jax.numpy as jnp
from jax.experimental import pallas as pl
from jax.experimental.pallas import tpu as pltpu

_C = 8
_BN = 32768
_CN = 8192
_NCORES = 2


def _onehot_bf16(x):
    m = jnp.max(x, axis=0, keepdims=True)
    return jnp.where(x == m, 1.0, 0.0).astype(jnp.bfloat16)


def _cm_kernel(xt_ref, xp_ref, out_ref):
    j = pl.program_id(1)

    @pl.when(j == 0)
    def _():
        out_ref[...] = jnp.zeros_like(out_ref)

    acc = jnp.zeros((_C, _C), jnp.float32)
    for c in range(_BN // _CN):
        sl = slice(c * _CN, (c + 1) * _CN)
        oh_t = _onehot_bf16(xt_ref[:, sl])
        oh_p = _onehot_bf16(xp_ref[:, sl])
        acc = acc + jax.lax.dot_general(
            oh_t, oh_p, (((1,), (1,)), ((), ())),
            preferred_element_type=jnp.float32,
        )
    out_ref[...] += acc


def kernel(y_true, y_pred):
    n = y_true.shape[0]
    xt = y_true.T
    xp = y_pred.T
    k = n // (_BN * _NCORES)
    g = pl.pallas_call(
        _cm_kernel,
        grid=(_NCORES, k),
        in_specs=[
            pl.BlockSpec((_C, _BN), lambda i, j: (0, i * k + j)),
            pl.BlockSpec((_C, _BN), lambda i, j: (0, i * k + j)),
        ],
        out_specs=pl.BlockSpec((None, _C, _C), lambda i, j: (i, 0, 0)),
        out_shape=jax.ShapeDtypeStruct((_NCORES, _C, _C), jnp.float32),
        compiler_params=pltpu.CompilerParams(
            dimension_semantics=("parallel", "arbitrary"),
        ),
    )(xt, xp)
    return g.sum(axis=0)


# speedup vs baseline: 53.1553x; 1.6066x over previous
import jax
import jax.numpy as jnp
from jax.experimental import pallas as pl
from jax.experimental.pallas import tpu as pltpu

_C = 8
_BN = 131072
_CN = 8192
_NCORES = 2


def _onehot_bf16(x):
    m = jnp.max(x, axis=0, keepdims=True)
    return jnp.where(x == m, 1.0, 0.0).astype(jnp.bfloat16)


def _cm_kernel(xt_ref, xp_ref, out_ref):
    j = pl.program_id(1)

    @pl.when(j == 0)
    def _():
        out_ref[...] = jnp.zeros_like(out_ref)

    acc = jnp.zeros((_C, _C), jnp.float32)
    for c in range(_BN // _CN):
        sl = slice(c * _CN, (c + 1) * _CN)
        oh_t = _onehot_bf16(xt_ref[:, sl])
        oh_p = _onehot_bf16(xp_ref[:, sl])
        acc = acc + jax.lax.dot_general(
            oh_t, oh_p, (((1,), (1,)), ((), ())),
            preferred_element_type=jnp.float32,
        )
    out_ref[...] += acc


def kernel(y_true, y_pred):
    n = y_true.shape[0]
    xt = y_true.T
    xp = y_pred.T
    k = n // (_BN * _NCORES)
    g = pl.pallas_call(
        _cm_kernel,
        grid=(_NCORES, k),
        in_specs=[
            pl.BlockSpec((_C, _BN), lambda i, j: (0, i * k + j)),
            pl.BlockSpec((_C, _BN), lambda i, j: (0, i * k + j)),
        ],
        out_specs=pl.BlockSpec((None, _C, _C), lambda i, j: (i, 0, 0)),
        out_shape=jax.ShapeDtypeStruct((_NCORES, _C, _C), jnp.float32),
        compiler_params=pltpu.CompilerParams(
            dimension_semantics=("parallel", "arbitrary"),
        ),
    )(xt, xp)
    return g.sum(axis=0)


# speedup vs baseline: 58.4119x; 1.0989x over previous
import jax
import jax.numpy as jnp
from jax.experimental import pallas as pl
from jax.experimental.pallas import tpu as pltpu

_C = 8
_BN = 262144
_CN = 8192
_NCORES = 2


def _onehot_bf16(x):
    m = jnp.max(x, axis=0, keepdims=True)
    return jnp.where(x == m, 1.0, 0.0).astype(jnp.bfloat16)


def _cm_kernel(xt_ref, xp_ref, out_ref):
    j = pl.program_id(1)

    @pl.when(j == 0)
    def _():
        out_ref[...] = jnp.zeros_like(out_ref)

    acc = jnp.zeros((_C, _C), jnp.float32)
    for c in range(_BN // _CN):
        sl = slice(c * _CN, (c + 1) * _CN)
        oh_t = _onehot_bf16(xt_ref[:, sl])
        oh_p = _onehot_bf16(xp_ref[:, sl])
        acc = acc + jax.lax.dot_general(
            oh_t, oh_p, (((1,), (1,)), ((), ())),
            preferred_element_type=jnp.float32,
        )
    out_ref[...] += acc


def kernel(y_true, y_pred):
    n = y_true.shape[0]
    xt = y_true.T
    xp = y_pred.T
    k = n // (_BN * _NCORES)
    g = pl.pallas_call(
        _cm_kernel,
        grid=(_NCORES, k),
        in_specs=[
            pl.BlockSpec((_C, _BN), lambda i, j: (0, i * k + j)),
            pl.BlockSpec((_C, _BN), lambda i, j: (0, i * k + j)),
        ],
        out_specs=pl.BlockSpec((None, _C, _C), lambda i, j: (i, 0, 0)),
        out_shape=jax.ShapeDtypeStruct((_NCORES, _C, _C), jnp.float32),
        compiler_params=pltpu.CompilerParams(
            dimension_semantics=("parallel", "arbitrary"),
        ),
    )(xt, xp)
    return g.sum(axis=0)
